# Initial kernel scaffold; baseline (speedup 1.0000x reference)
import jax
import jax.numpy as jnp
from jax.experimental import pallas as pl


def kernel(context, target, i_table, o_table):
    raise NotImplementedError("write your pallas kernel here")


# speedup vs baseline: 1.0966x; 1.0966x over previous
import functools

import jax
import jax.numpy as jnp
from jax import lax
from jax.experimental import pallas as pl
from jax.experimental.pallas import tpu as pltpu
from jax.experimental.pallas import tpu_sc as plsc

EMB_COUNT = 1000000
EMB_DIM = 32
NEG_K = 20
CTX_LEN = 20
BATCH_N = 16384

NUM_CORES = 2
NUM_SUBCORES = 16
NW = NUM_CORES * NUM_SUBCORES
BPW = BATCH_N // NW
ROWS_PW = BPW * CTX_LEN
CHUNK = 2048


def _sc_gather(ctx_idx, neg_idx, tgt_idx, i_table, o_table):
    mesh = plsc.VectorSubcoreMesh(core_axis_name="c", subcore_axis_name="s")

    @functools.partial(
        pl.kernel,
        mesh=mesh,
        compiler_params=pltpu.CompilerParams(use_tc_tiling_on_sc=False),
        out_type=(
            jax.ShapeDtypeStruct((BATCH_N * CTX_LEN, EMB_DIM), jnp.float32),
            jax.ShapeDtypeStruct((BATCH_N * NEG_K, EMB_DIM), jnp.float32),
            jax.ShapeDtypeStruct((BATCH_N, EMB_DIM), jnp.float32),
        ),
        scratch_types=[
            pltpu.VMEM((CHUNK,), jnp.int32),
            pltpu.VMEM((CHUNK, EMB_DIM), jnp.float32),
            pltpu.VMEM((BPW,), jnp.int32),
            pltpu.VMEM((BPW, EMB_DIM), jnp.float32),
            pltpu.SemaphoreType.DMA,
        ],
    )
    def k(ctx_hbm, neg_hbm, tgt_hbm, it_hbm, ot_hbm, octx, oneg, otgt,
          idx_v, rows_v, tidx_v, trows_v, sem):
        wid = lax.axis_index("s") * NUM_CORES + lax.axis_index("c")
        base = wid * ROWS_PW
        for t in range(ROWS_PW // CHUNK):
            off = base + t * CHUNK
            pltpu.sync_copy(ctx_hbm.at[pl.ds(off, CHUNK)], idx_v)
            pltpu.async_copy(ot_hbm.at[idx_v], rows_v, sem).wait()
            pltpu.sync_copy(rows_v, octx.at[pl.ds(off, CHUNK)])
            pltpu.sync_copy(neg_hbm.at[pl.ds(off, CHUNK)], idx_v)
            pltpu.async_copy(ot_hbm.at[idx_v], rows_v, sem).wait()
            pltpu.sync_copy(rows_v, oneg.at[pl.ds(off, CHUNK)])
        tb = wid * BPW
        pltpu.sync_copy(tgt_hbm.at[pl.ds(tb, BPW)], tidx_v)
        pltpu.async_copy(it_hbm.at[tidx_v], trows_v, sem).wait()
        pltpu.sync_copy(trows_v, otgt.at[pl.ds(tb, BPW)])

    return k(ctx_idx, neg_idx, tgt_idx, i_table, o_table)


def _log_sigmoid(x):
    return jnp.minimum(x, 0.0) - jnp.log1p(jnp.exp(-jnp.abs(x)))


_TC_BB = 256


def _tc_loss_body(ctx_ref, neg_ref, tgt_ref, out_ref):
    i = pl.program_id(0)
    tgt = tgt_ref[...]
    pos_s = jnp.sum(ctx_ref[...] * tgt[:, None, :], axis=2)
    neg_s = jnp.sum(neg_ref[...] * tgt[:, None, :], axis=2)
    pos = jnp.mean(_log_sigmoid(pos_s), axis=1)
    neg = jnp.sum(_log_sigmoid(-neg_s), axis=1)
    part = jnp.sum(pos + neg)

    @pl.when(i == 0)
    def _():
        out_ref[0, 0] = 0.0

    out_ref[0, 0] += part


def _tc_loss(ctx_emb, neg_emb, tgt_emb):
    grid = (BATCH_N // _TC_BB,)
    acc = pl.pallas_call(
        _tc_loss_body,
        grid=grid,
        in_specs=[
            pl.BlockSpec((_TC_BB, CTX_LEN, EMB_DIM), lambda i: (i, 0, 0)),
            pl.BlockSpec((_TC_BB, NEG_K, EMB_DIM), lambda i: (i, 0, 0)),
            pl.BlockSpec((_TC_BB, EMB_DIM), lambda i: (i, 0)),
        ],
        out_specs=pl.BlockSpec(memory_space=pltpu.SMEM),
        out_shape=jax.ShapeDtypeStruct((1, 1), jnp.float32),
    )(ctx_emb, neg_emb, tgt_emb)
    return -acc[0, 0] / BATCH_N


def kernel(context, target, i_table, o_table):
    b = context.shape[0]
    neg_samples = jax.random.randint(
        jax.random.key(12345), (b, NEG_K), 0, EMB_COUNT - 1)
    ctx_idx = context.astype(jnp.int32).reshape(-1)
    neg_idx = neg_samples.astype(jnp.int32).reshape(-1)
    tgt_idx = target.astype(jnp.int32)
    ctx_emb, neg_emb, tgt_emb = _sc_gather(
        ctx_idx, neg_idx, tgt_idx, i_table, o_table)
    return _tc_loss(
        ctx_emb.reshape(BATCH_N, CTX_LEN, EMB_DIM),
        neg_emb.reshape(BATCH_N, NEG_K, EMB_DIM),
        tgt_emb,
    )


# speedup vs baseline: 1.6896x; 1.5408x over previous
import functools

import jax
import jax.numpy as jnp
from jax import lax
from jax.experimental import pallas as pl
from jax.experimental.pallas import tpu as pltpu
from jax.experimental.pallas import tpu_sc as plsc

EMB_COUNT = 1000000
EMB_DIM = 32
NEG_K = 20
CTX_LEN = 20
BATCH_N = 16384

NUM_CORES = 2
NUM_SUBCORES = 16
NW = NUM_CORES * NUM_SUBCORES
BPW = BATCH_N // NW
CB = 64
NCHUNK = BPW // CB
CROWS = CB * CTX_LEN


def _sc_scores(ctx_idx, neg_idx, tgt_idx, i_table, o_table):
    mesh = plsc.VectorSubcoreMesh(core_axis_name="c", subcore_axis_name="s")

    @functools.partial(
        pl.kernel,
        mesh=mesh,
        compiler_params=pltpu.CompilerParams(
            use_tc_tiling_on_sc=False, needs_layout_passes=False),
        out_type=(
            jax.ShapeDtypeStruct((BATCH_N * CTX_LEN,), jnp.float32),
            jax.ShapeDtypeStruct((BATCH_N * NEG_K,), jnp.float32),
        ),
        scratch_types=[
            pltpu.VMEM((CROWS,), jnp.int32),
            pltpu.VMEM((CROWS,), jnp.int32),
            pltpu.VMEM((CB,), jnp.int32),
            pltpu.VMEM((CROWS, EMB_DIM), jnp.float32),
            pltpu.VMEM((CROWS, EMB_DIM), jnp.float32),
            pltpu.VMEM((CB, EMB_DIM), jnp.float32),
            pltpu.VMEM((CROWS,), jnp.float32),
            pltpu.VMEM((CROWS,), jnp.float32),
            pltpu.SemaphoreType.DMA,
        ],
    )
    def k(ctx_hbm, neg_hbm, tgt_hbm, it_hbm, ot_hbm, ps_hbm, ns_hbm,
          cidx_v, nidx_v, tidx_v, crows_v, nrows_v, trows_v, ps_v, ns_v, sem):
        wid = lax.axis_index("s") * NUM_CORES + lax.axis_index("c")
        lane = lax.iota(jnp.int32, 16)
        masks = [lane == l for l in range(16)]

        def chunk_body(t, carry0):
            roff = (wid * NCHUNK + t) * CROWS
            boff = (wid * NCHUNK + t) * CB
            pltpu.sync_copy(ctx_hbm.at[pl.ds(roff, CROWS)], cidx_v)
            pltpu.sync_copy(neg_hbm.at[pl.ds(roff, CROWS)], nidx_v)
            pltpu.sync_copy(tgt_hbm.at[pl.ds(boff, CB)], tidx_v)
            g1 = pltpu.async_copy(ot_hbm.at[cidx_v], crows_v, sem)
            g2 = pltpu.async_copy(ot_hbm.at[nidx_v], nrows_v, sem)
            g3 = pltpu.async_copy(it_hbm.at[tidx_v], trows_v, sem)
            g1.wait()
            g2.wait()
            g3.wait()

            def group_body(g, carry1):
                t0 = t1 = None
                pacc = nacc = None
                for j in range(4 * CTX_LEN):
                    if j % CTX_LEN == 0:
                        b = g * 4 + (j // CTX_LEN)
                        t0 = trows_v[b, pl.ds(0, 16)]
                        t1 = trows_v[b, pl.ds(16, 16)]
                    i = g * (4 * CTX_LEN) + j
                    v, l = j // 16, j % 16
                    p = (crows_v[i, pl.ds(0, 16)] * t0
                         + crows_v[i, pl.ds(16, 16)] * t1)
                    q = (nrows_v[i, pl.ds(0, 16)] * t0
                         + nrows_v[i, pl.ds(16, 16)] * t1)
                    ps = jnp.sum(p)
                    ns = jnp.sum(q)
                    if l == 0:
                        pacc = jnp.where(masks[0], ps, 0.0)
                        nacc = jnp.where(masks[0], ns, 0.0)
                    else:
                        pacc = jnp.where(masks[l], ps, pacc)
                        nacc = jnp.where(masks[l], ns, nacc)
                    if l == 15:
                        off = g * (4 * CTX_LEN) + v * 16
                        ps_v[pl.ds(off, 16)] = pacc
                        ns_v[pl.ds(off, 16)] = nacc
                return carry1

            lax.fori_loop(0, CROWS // (4 * CTX_LEN), group_body, 0)
            pltpu.sync_copy(ps_v, ps_hbm.at[pl.ds(roff, CROWS)])
            pltpu.sync_copy(ns_v, ns_hbm.at[pl.ds(roff, CROWS)])
            return carry0

        lax.fori_loop(0, NCHUNK, chunk_body, 0)

    return k(ctx_idx, neg_idx, tgt_idx, i_table, o_table)


def _log_sigmoid(x):
    return jnp.minimum(x, 0.0) - jnp.log1p(jnp.exp(-jnp.abs(x)))


def _tc_loss_body(ps_ref, ns_ref, out_ref):
    out_ref[0] = jnp.sum(_log_sigmoid(ps_ref[...]))
    out_ref[1] = jnp.sum(_log_sigmoid(-ns_ref[...]))


def _tc_loss(pos_s, neg_s):
    n = BATCH_N * CTX_LEN
    acc = pl.pallas_call(
        _tc_loss_body,
        in_specs=[
            pl.BlockSpec((n // 128, 128), lambda: (0, 0)),
            pl.BlockSpec((n // 128, 128), lambda: (0, 0)),
        ],
        out_specs=pl.BlockSpec(memory_space=pltpu.SMEM),
        out_shape=jax.ShapeDtypeStruct((2,), jnp.float32),
    )(pos_s.reshape(n // 128, 128), neg_s.reshape(n // 128, 128))
    return -(acc[0] / CTX_LEN + acc[1]) / BATCH_N


def kernel(context, target, i_table, o_table):
    b = context.shape[0]
    neg_samples = jax.random.randint(
        jax.random.key(12345), (b, NEG_K), 0, EMB_COUNT - 1)
    ctx_idx = context.astype(jnp.int32).reshape(-1)
    neg_idx = neg_samples.astype(jnp.int32).reshape(-1)
    tgt_idx = target.astype(jnp.int32)
    pos_s, neg_s = _sc_scores(ctx_idx, neg_idx, tgt_idx, i_table, o_table)
    return _tc_loss(pos_s, neg_s)
